# baseline (device time: 90530 ns/iter reference)
import jax
import jax.numpy as jnp
from jax import lax
from jax.experimental import pallas as pl
from jax.experimental.pallas import tpu as pltpu

N_DEV = 4
SQ = 2048
D_MODEL = 1024
H_LOC = 8
DH = 128
BLK = 64
N_PHASE = 4
GROUPS = SQ // (BLK * N_PHASE)
P_SEQ = GROUPS * BLK
SCALE = 0.08838834764831843
QROWS = P_SEQ // N_DEV


def _body(
    my_ref, x_ref, wq_ref, k_ref, v_ref, wo_ref, out_ref,
    ctx_acc, part_ref, rs_buf, ag_src, ag_buf, xg_ref, wo_bf,
    rs_send, rs_recv, ag_send, ag_recv,
):
    p = pl.program_id(0)
    h = pl.program_id(1)
    my = my_ref[0]

    @pl.when((p == 0) & (h == 0))
    def _barrier():
        barrier = pltpu.get_barrier_semaphore()
        for k in (1, 2, 3):
            pl.semaphore_signal(
                barrier, inc=1, device_id=((my + k) % N_DEV,),
                device_id_type=pl.DeviceIdType.MESH,
            )
        pl.semaphore_wait(barrier, 3)
        wo_bf[:, :] = wo_ref[...].astype(jnp.bfloat16)

    @pl.when(h == 0)
    def _gather_x():
        for g in range(GROUPS):
            xg_ref[pl.ds(g * BLK, BLK), :] = x_ref[
                pl.ds(g * N_PHASE * BLK + p * BLK, BLK), :
            ].astype(jnp.bfloat16)

    q = jnp.dot(
        xg_ref[...], wq_ref[...].astype(jnp.bfloat16),
        preferred_element_type=jnp.float32,
    )
    s = lax.dot_general(
        q.astype(jnp.bfloat16), k_ref[0, 0],
        (((1,), (1,)), ((), ())),
        preferred_element_type=jnp.float32,
    ) * SCALE
    w = jnp.exp(s)
    denom = jnp.sum(w, axis=-1, keepdims=True)
    ctx = jnp.dot(
        w.astype(jnp.bfloat16), v_ref[0, 0],
        preferred_element_type=jnp.float32,
    ) / denom
    ctx_acc[:, pl.ds(h * DH, DH)] = ctx.astype(jnp.bfloat16)

    def _finalize_phase(pp):
        red = part_ref[pp, pl.ds(my * QROWS, QROWS), :].astype(jnp.float32)
        for k in (1, 2, 3):
            rr = pltpu.make_async_remote_copy(
                src_ref=rs_buf.at[pp * 3 + k - 1],
                dst_ref=rs_buf.at[pp * 3 + k - 1],
                send_sem=rs_send.at[pp * 3 + k - 1],
                recv_sem=rs_recv.at[pp * 3 + k - 1],
                device_id=(my,), device_id_type=pl.DeviceIdType.MESH,
            )
            rr.wait_recv()
            red = red + rs_buf[pp * 3 + k - 1, :, :].astype(jnp.float32)
        ag_src[pp, :, :] = red.astype(jnp.bfloat16)
        for k in (1, 2, 3):
            t = (my + k) % N_DEV
            r = pltpu.make_async_remote_copy(
                src_ref=ag_src.at[pp],
                dst_ref=ag_buf.at[pp * 3 + k - 1],
                send_sem=ag_send.at[pp * 3 + k - 1],
                recv_sem=ag_recv.at[pp * 3 + k - 1],
                device_id=(t,),
                device_id_type=pl.DeviceIdType.MESH,
            )
            r.start()
        for i in range(2):
            out_ref[0, pl.ds(my * 512 + i * 256 + pp * BLK, BLK), :] = (
                red[i * BLK:(i + 1) * BLK, :]
            )

    @pl.when(h == H_LOC - 1)
    def _phase_end():
        for pp in range(N_PHASE):
            @pl.when(p == pp)
            def _(pp=pp):
                for k in (1, 2, 3, 0):
                    q = (my + k) % N_DEV
                    qpart = jnp.dot(
                        ctx_acc[pl.ds(q * QROWS, QROWS), :], wo_bf[...],
                        preferred_element_type=jnp.float32,
                    )
                    part_ref[pp, pl.ds(q * QROWS, QROWS), :] = (
                        qpart.astype(jnp.bfloat16)
                    )
                    if k == 0:
                        continue
                    r = pltpu.make_async_remote_copy(
                        src_ref=part_ref.at[pp, pl.ds(q * QROWS, QROWS)],
                        dst_ref=rs_buf.at[pp * 3 + k - 1],
                        send_sem=rs_send.at[pp * 3 + k - 1],
                        recv_sem=rs_recv.at[pp * 3 + k - 1],
                        device_id=(q,),
                        device_id_type=pl.DeviceIdType.MESH,
                    )
                    r.start()
                if pp >= 1:
                    _finalize_phase(pp - 1)
                if pp == N_PHASE - 1:
                    _finalize_phase(pp)
                    for qp in range(N_PHASE):
                        for k in (1, 2, 3):
                            rr = pltpu.make_async_remote_copy(
                                src_ref=ag_buf.at[qp * 3 + k - 1],
                                dst_ref=ag_buf.at[qp * 3 + k - 1],
                                send_sem=ag_send.at[qp * 3 + k - 1],
                                recv_sem=ag_recv.at[qp * 3 + k - 1],
                                device_id=(my,),
                                device_id_type=pl.DeviceIdType.MESH,
                            )
                            rr.wait_recv()
                            sd = (my + N_DEV - k) % N_DEV
                            chunk = ag_buf[qp * 3 + k - 1, :, :].astype(
                                jnp.float32
                            )
                            for i in range(2):
                                out_ref[
                                    0,
                                    pl.ds(sd * 512 + i * 256 + qp * BLK, BLK),
                                    :,
                                ] = chunk[i * BLK:(i + 1) * BLK, :]
                    for s in range(3 * N_PHASE):
                        pltpu.make_async_remote_copy(
                            src_ref=rs_buf.at[s], dst_ref=rs_buf.at[s],
                            send_sem=rs_send.at[s], recv_sem=rs_recv.at[s],
                            device_id=(my,),
                            device_id_type=pl.DeviceIdType.MESH,
                        ).wait_send()
                        pltpu.make_async_remote_copy(
                            src_ref=ag_buf.at[s], dst_ref=ag_buf.at[s],
                            send_sem=ag_send.at[s], recv_sem=ag_recv.at[s],
                            device_id=(my,),
                            device_id_type=pl.DeviceIdType.MESH,
                        ).wait_send()


def kernel(x, Wq, K_ext, V_ext, Wo):
    my = lax.axis_index("i")
    f_loc = H_LOC * DH
    my_arr = jnp.full((1,), my, dtype=jnp.int32)

    kb = K_ext[0].astype(jnp.bfloat16)
    kp = kb.reshape(GROUPS, N_PHASE, BLK, H_LOC, DH)
    kp = kp.transpose(3, 1, 0, 2, 4).reshape(H_LOC, N_PHASE, P_SEQ, DH)
    vb = V_ext[0].astype(jnp.bfloat16)
    vp = vb.reshape(GROUPS, N_PHASE, BLK, H_LOC, DH)
    vp = vp.transpose(3, 1, 0, 2, 4).reshape(H_LOC, N_PHASE, P_SEQ, DH)

    grid_spec = pltpu.PrefetchScalarGridSpec(
        num_scalar_prefetch=1,
        grid=(N_PHASE, H_LOC),
        in_specs=[
            pl.BlockSpec((SQ, D_MODEL), lambda p, h, m: (0, 0)),
            pl.BlockSpec((D_MODEL, DH), lambda p, h, m: (0, m[0] * H_LOC + h)),
            pl.BlockSpec((1, 1, P_SEQ, DH), lambda p, h, m: (h, p, 0, 0)),
            pl.BlockSpec((1, 1, P_SEQ, DH), lambda p, h, m: (h, p, 0, 0)),
            pl.BlockSpec((f_loc, D_MODEL), lambda p, h, m: (m[0], 0)),
        ],
        out_specs=pl.BlockSpec((1, SQ, D_MODEL), lambda p, h, m: (0, 0, 0)),
        scratch_shapes=[
            pltpu.VMEM((P_SEQ, f_loc), jnp.bfloat16),
            pltpu.VMEM((N_PHASE, P_SEQ, D_MODEL), jnp.bfloat16),
            pltpu.VMEM((3 * N_PHASE, QROWS, D_MODEL), jnp.bfloat16),
            pltpu.VMEM((N_PHASE, QROWS, D_MODEL), jnp.bfloat16),
            pltpu.VMEM((3 * N_PHASE, QROWS, D_MODEL), jnp.bfloat16),
            pltpu.VMEM((P_SEQ, D_MODEL), jnp.bfloat16),
            pltpu.VMEM((f_loc, D_MODEL), jnp.bfloat16),
            pltpu.SemaphoreType.DMA((3 * N_PHASE,)),
            pltpu.SemaphoreType.DMA((3 * N_PHASE,)),
            pltpu.SemaphoreType.DMA((3 * N_PHASE,)),
            pltpu.SemaphoreType.DMA((3 * N_PHASE,)),
        ],
    )

    out = pl.pallas_call(
        _body,
        grid_spec=grid_spec,
        out_shape=jax.ShapeDtypeStruct((1, SQ, D_MODEL), jnp.float32),
        compiler_params=pltpu.CompilerParams(collective_id=0),
    )(my_arr, x[0], Wq, kp, vp, Wo)

    return out


# device time: 88014 ns/iter; 1.0286x vs baseline; 1.0286x over previous
import jax
import jax.numpy as jnp
from jax import lax
from jax.experimental import pallas as pl
from jax.experimental.pallas import tpu as pltpu

N_DEV = 4
SQ = 2048
D_MODEL = 1024
H_LOC = 8
DH = 128
BLK = 64
N_PHASE = 4
GROUPS = SQ // (BLK * N_PHASE)
P_SEQ = GROUPS * BLK
SCALE = 0.08838834764831843
QROWS = P_SEQ // N_DEV


def _body(
    my_ref, x_ref, wq_ref, k_ref, v_ref, wo_ref, out_ref,
    ctx_acc, part_ref, rs_buf, ag_src, ag_buf, xg_ref, wo_bf,
    rs_send, rs_recv, ag_send, ag_recv,
):
    p = pl.program_id(0)
    h = pl.program_id(1)
    my = my_ref[0]

    @pl.when((p == 0) & (h == 0))
    def _barrier():
        barrier = pltpu.get_barrier_semaphore()
        for k in (1, 2, 3):
            pl.semaphore_signal(
                barrier, inc=1, device_id=((my + k) % N_DEV,),
                device_id_type=pl.DeviceIdType.MESH,
            )
        pl.semaphore_wait(barrier, 3)
        wo_bf[:, :] = wo_ref[...].astype(jnp.bfloat16)

    @pl.when(h == 0)
    def _gather_x():
        for g in range(GROUPS):
            xg_ref[pl.ds(g * BLK, BLK), :] = x_ref[
                pl.ds(g * N_PHASE * BLK + p * BLK, BLK), :
            ].astype(jnp.bfloat16)

    q = jnp.dot(
        xg_ref[...], wq_ref[...].astype(jnp.bfloat16),
        preferred_element_type=jnp.float32,
    )
    s = lax.dot_general(
        q.astype(jnp.bfloat16), k_ref[0, 0],
        (((1,), (1,)), ((), ())),
        preferred_element_type=jnp.float32,
    ) * SCALE
    w = jnp.exp(s)
    denom = jnp.sum(w, axis=-1, keepdims=True)
    ctx = jnp.dot(
        w.astype(jnp.bfloat16), v_ref[0, 0],
        preferred_element_type=jnp.float32,
    ) / denom
    ctx_acc[:, pl.ds(h * DH, DH)] = ctx.astype(jnp.bfloat16)

    def _finalize_phase(pp):
        red = part_ref[pp, pl.ds(my * QROWS, QROWS), :].astype(jnp.float32)
        for k in (1, 2, 3):
            rr = pltpu.make_async_remote_copy(
                src_ref=rs_buf.at[pp * 3 + k - 1],
                dst_ref=rs_buf.at[pp * 3 + k - 1],
                send_sem=rs_send.at[pp * 3 + k - 1],
                recv_sem=rs_recv.at[pp * 3 + k - 1],
                device_id=(my,), device_id_type=pl.DeviceIdType.MESH,
            )
            rr.wait_recv()
            red = red + rs_buf[pp * 3 + k - 1, :, :].astype(jnp.float32)
        ag_src[pp, :, :] = red.astype(jnp.bfloat16)
        for k in (1, 2, 3):
            t = (my + k) % N_DEV
            r = pltpu.make_async_remote_copy(
                src_ref=ag_src.at[pp],
                dst_ref=ag_buf.at[pp * 3 + k - 1],
                send_sem=ag_send.at[pp * 3 + k - 1],
                recv_sem=ag_recv.at[pp * 3 + k - 1],
                device_id=(t,),
                device_id_type=pl.DeviceIdType.MESH,
            )
            r.start()
        for i in range(2):
            out_ref[0, pl.ds(my * 512 + i * 256 + pp * BLK, BLK), :] = (
                red[i * BLK:(i + 1) * BLK, :]
            )

    @pl.when(h == H_LOC - 1)
    def _phase_end():
        partial = jnp.dot(
            ctx_acc[...], wo_bf[...], preferred_element_type=jnp.float32
        )
        part_ref[p, :, :] = partial.astype(jnp.bfloat16)
        for pp in range(N_PHASE):
            @pl.when(p == pp)
            def _(pp=pp):
                for k in (1, 2, 3):
                    q = (my + k) % N_DEV
                    r = pltpu.make_async_remote_copy(
                        src_ref=part_ref.at[pp, pl.ds(q * QROWS, QROWS)],
                        dst_ref=rs_buf.at[pp * 3 + k - 1],
                        send_sem=rs_send.at[pp * 3 + k - 1],
                        recv_sem=rs_recv.at[pp * 3 + k - 1],
                        device_id=(q,),
                        device_id_type=pl.DeviceIdType.MESH,
                    )
                    r.start()
                if pp >= 1:
                    _finalize_phase(pp - 1)
                if pp == N_PHASE - 1:
                    _finalize_phase(pp)
                    for qp in range(N_PHASE):
                        for k in (1, 2, 3):
                            rr = pltpu.make_async_remote_copy(
                                src_ref=ag_buf.at[qp * 3 + k - 1],
                                dst_ref=ag_buf.at[qp * 3 + k - 1],
                                send_sem=ag_send.at[qp * 3 + k - 1],
                                recv_sem=ag_recv.at[qp * 3 + k - 1],
                                device_id=(my,),
                                device_id_type=pl.DeviceIdType.MESH,
                            )
                            rr.wait_recv()
                            sd = (my + N_DEV - k) % N_DEV
                            chunk = ag_buf[qp * 3 + k - 1, :, :].astype(
                                jnp.float32
                            )
                            for i in range(2):
                                out_ref[
                                    0,
                                    pl.ds(sd * 512 + i * 256 + qp * BLK, BLK),
                                    :,
                                ] = chunk[i * BLK:(i + 1) * BLK, :]
                    for s in range(3 * N_PHASE):
                        pltpu.make_async_remote_copy(
                            src_ref=rs_buf.at[s], dst_ref=rs_buf.at[s],
                            send_sem=rs_send.at[s], recv_sem=rs_recv.at[s],
                            device_id=(my,),
                            device_id_type=pl.DeviceIdType.MESH,
                        ).wait_send()
                        pltpu.make_async_remote_copy(
                            src_ref=ag_buf.at[s], dst_ref=ag_buf.at[s],
                            send_sem=ag_send.at[s], recv_sem=ag_recv.at[s],
                            device_id=(my,),
                            device_id_type=pl.DeviceIdType.MESH,
                        ).wait_send()


def kernel(x, Wq, K_ext, V_ext, Wo):
    my = lax.axis_index("i")
    f_loc = H_LOC * DH
    my_arr = jnp.full((1,), my, dtype=jnp.int32)

    kb = K_ext[0].astype(jnp.bfloat16)
    kp = kb.reshape(GROUPS, N_PHASE, BLK, H_LOC, DH)
    kp = kp.transpose(3, 1, 0, 2, 4).reshape(H_LOC, N_PHASE, P_SEQ, DH)
    vb = V_ext[0].astype(jnp.bfloat16)
    vp = vb.reshape(GROUPS, N_PHASE, BLK, H_LOC, DH)
    vp = vp.transpose(3, 1, 0, 2, 4).reshape(H_LOC, N_PHASE, P_SEQ, DH)

    grid_spec = pltpu.PrefetchScalarGridSpec(
        num_scalar_prefetch=1,
        grid=(N_PHASE, H_LOC),
        in_specs=[
            pl.BlockSpec((SQ, D_MODEL), lambda p, h, m: (0, 0)),
            pl.BlockSpec((D_MODEL, DH), lambda p, h, m: (0, m[0] * H_LOC + h)),
            pl.BlockSpec((1, 1, P_SEQ, DH), lambda p, h, m: (h, p, 0, 0)),
            pl.BlockSpec((1, 1, P_SEQ, DH), lambda p, h, m: (h, p, 0, 0)),
            pl.BlockSpec((f_loc, D_MODEL), lambda p, h, m: (m[0], 0)),
        ],
        out_specs=pl.BlockSpec((1, SQ, D_MODEL), lambda p, h, m: (0, 0, 0)),
        scratch_shapes=[
            pltpu.VMEM((P_SEQ, f_loc), jnp.bfloat16),
            pltpu.VMEM((N_PHASE, P_SEQ, D_MODEL), jnp.bfloat16),
            pltpu.VMEM((3 * N_PHASE, QROWS, D_MODEL), jnp.bfloat16),
            pltpu.VMEM((N_PHASE, QROWS, D_MODEL), jnp.bfloat16),
            pltpu.VMEM((3 * N_PHASE, QROWS, D_MODEL), jnp.bfloat16),
            pltpu.VMEM((P_SEQ, D_MODEL), jnp.bfloat16),
            pltpu.VMEM((f_loc, D_MODEL), jnp.bfloat16),
            pltpu.SemaphoreType.DMA((3 * N_PHASE,)),
            pltpu.SemaphoreType.DMA((3 * N_PHASE,)),
            pltpu.SemaphoreType.DMA((3 * N_PHASE,)),
            pltpu.SemaphoreType.DMA((3 * N_PHASE,)),
        ],
    )

    out = pl.pallas_call(
        _body,
        grid_spec=grid_spec,
        out_shape=jax.ShapeDtypeStruct((1, SQ, D_MODEL), jnp.float32),
        compiler_params=pltpu.CompilerParams(collective_id=0),
    )(my_arr, x[0], Wq, kp, vp, Wo)

    return out
